# Initial kernel scaffold; baseline (speedup 1.0000x reference)
import jax
import jax.numpy as jnp
from jax.experimental import pallas as pl


def kernel(x, adj, W1, b1, W2, b2, W3, b3, scale):
    raise NotImplementedError("write your pallas kernel here")


# speedup vs baseline: 4.5558x; 4.5558x over previous
import functools

import jax
import jax.numpy as jnp
from jax import lax
from jax.experimental import pallas as pl
from jax.experimental.pallas import tpu as pltpu
from jax.experimental.pallas import tpu_sc as plsc

N = 4096
E = 65536
IN = 128
HID = 256
OUT = 16
R = 2.0
T = 1.0
EPS = 1e-5

NC = 2
NS = 16
CH = 128
NW = NC * NS
NCHUNK = E // (NW * CH)
RPT = N // NS

D1 = 272
D2 = 256
D3 = 32

BR = 512


def _seg_sum_partials(m, src, dst, d):
    mesh = plsc.VectorSubcoreMesh(
        core_axis_name="c", subcore_axis_name="s",
        num_cores=NC, num_subcores=NS)

    def body(m_hbm, src_hbm, dst_hbm, zeros_hbm, out_hbm,
             src_v, dst_v, rows_v, agg_sh, sem):
        cid = lax.axis_index("c")
        sid = lax.axis_index("s")
        w = cid * NS + sid
        pltpu.sync_copy(zeros_hbm, agg_sh.at[pl.ds(sid * RPT, RPT)])
        pltpu.sync_copy(src_hbm.at[w], src_v)
        pltpu.sync_copy(dst_hbm.at[w], dst_v)
        plsc.subcore_barrier()

        def chunk(j, carry):
            pltpu.async_copy(m_hbm.at[src_v.at[j]], rows_v, sem).wait()
            pltpu.sync_copy(rows_v, agg_sh.at[dst_v.at[j]], add=True)
            return carry

        lax.fori_loop(0, NCHUNK, chunk, 0)
        plsc.subcore_barrier()
        pltpu.sync_copy(agg_sh.at[pl.ds(sid * RPT, RPT)],
                        out_hbm.at[cid, pl.ds(sid * RPT, RPT)])

    f = pl.kernel(
        body,
        out_type=jax.ShapeDtypeStruct((NC, N, d), jnp.float32),
        mesh=mesh,
        compiler_params=pltpu.CompilerParams(use_tc_tiling_on_sc=False),
        scratch_types=[
            pltpu.VMEM((NCHUNK, CH), jnp.int32),
            pltpu.VMEM((NCHUNK, CH), jnp.int32),
            pltpu.VMEM((CH, d), jnp.float32),
            pltpu.VMEM_SHARED((N, d), jnp.float32),
            pltpu.SemaphoreType.DMA,
        ],
    )
    zeros = jnp.zeros((RPT, d), jnp.float32)
    return f(m, src, dst, zeros)


def _log0(v):
    nrm = jnp.maximum(jnp.sqrt(jnp.sum(v * v, axis=1, keepdims=True)), EPS)
    en = jnp.exp(nrm)
    eni = 1.0 / en
    c = 0.5 * (en + eni)
    s = 0.5 * (en - eni)
    xs = (s / nrm) * v
    nrm2 = jnp.maximum(jnp.sqrt(jnp.sum(xs * xs, axis=1, keepdims=True)), EPS)
    y = jnp.maximum(c, 1.0 + EPS)
    dfac = jnp.log(y + jnp.sqrt(y + 1.0) * jnp.sqrt(y - 1.0))
    return (dfac / nrm2) * xs


def _enc_in_body(x_ref, w_ref, b_ref, o_ref):
    h = _log0(x_ref[...])
    m = jnp.dot(h, w_ref[...], preferred_element_type=jnp.float32) + b_ref[...]
    br = m.shape[0]
    o_ref[...] = jnp.concatenate(
        [m, jnp.ones((br, 1), jnp.float32), jnp.zeros((br, 15), jnp.float32)],
        axis=1)


def _mid1_body(m_ref, a0_ref, a1_ref, w_ref, b_ref, o_ref, deg_ref):
    agg = a0_ref[0] + a1_ref[0]
    deg = jnp.maximum(agg[:, HID:HID + 1], 1.0)
    out = m_ref[:, :HID] + agg[:, :HID] / deg
    h = _log0(out[:, 1:])
    m2 = jnp.dot(h, w_ref[...], preferred_element_type=jnp.float32) + b_ref[...]
    o_ref[...] = m2
    deg_ref[...] = jnp.broadcast_to(deg, (deg.shape[0], 8))


def _mid2_body(m_ref, a0_ref, a1_ref, deg_ref, w_ref, b_ref, o_ref):
    agg = a0_ref[0] + a1_ref[0]
    deg = deg_ref[:, 0:1]
    out = jnp.maximum(m_ref[...] + agg / deg, 0.0)
    h = _log0(out[:, 1:])
    o_ref[...] = (jnp.dot(h, w_ref[...], preferred_element_type=jnp.float32)
                  + b_ref[...])


def _fin_body(m_ref, a0_ref, a1_ref, deg_ref, sc_ref, z_ref):
    agg = a0_ref[0] + a1_ref[0]
    deg = deg_ref[:, 0:1]
    out = jnp.maximum(m_ref[...] + agg / deg, 0.0)
    v = out[:, 1:OUT + 1]
    nrm = jnp.maximum(jnp.sqrt(jnp.sum(v * v, axis=1, keepdims=True)), EPS)
    en = jnp.exp(nrm)
    eni = 1.0 / en
    x0 = 0.5 * (en + eni)
    xs = (0.5 * (en - eni) / nrm) * v
    p = xs / (x0 + 1.0)
    pn = jnp.maximum(jnp.sqrt(jnp.sum(p * p, axis=1, keepdims=True)), 1e-12)
    s = jnp.clip(sc_ref[0, 0], 0.01, 0.999)
    ph = (p / pn) * s
    sq = jnp.sum(ph * ph, axis=1, keepdims=True)
    denom = jnp.maximum(1.0 - sq, EPS)
    z0 = (1.0 + sq) / denom
    zs = 2.0 * ph / denom
    br = z0.shape[0]
    z_ref[...] = jnp.concatenate(
        [z0, zs, jnp.zeros((br, D3 - OUT - 1), jnp.float32)], axis=1)


def _dec_body(zi_ref, zj_ref, o_ref):
    zi = zi_ref[...]
    zj = zj_ref[...]
    col = lax.broadcasted_iota(jnp.int32, (1, D3), 1)
    sgn = jnp.where(col == 0, -1.0, jnp.where(col <= OUT, 1.0, 0.0))
    zim = zi * sgn
    ip = lax.dot_general(zim, zj, (((1,), (1,)), ((), ())),
                         preferred_element_type=jnp.float32)
    y = jnp.maximum(-ip, 1.0 + EPS)
    dh = jnp.log(y + jnp.sqrt(y + 1.0) * jnp.sqrt(y - 1.0))
    t = (R - dh * dh) / T
    p1 = 1.0 / (1.0 + jnp.exp(-t))
    o_ref[...] = 1.0 / (1.0 + jnp.exp(-p1))


def _row_call(body, widths_in, width_out, extra_outs=()):
    grid = (N // BR,)
    in_specs = []
    for kind, w in widths_in:
        if kind == "rows":
            in_specs.append(pl.BlockSpec((BR, w), lambda i: (i, 0)))
        elif kind == "part0":
            in_specs.append(pl.BlockSpec((1, BR, w), lambda i: (0, i, 0)))
        elif kind == "part1":
            in_specs.append(pl.BlockSpec((1, BR, w), lambda i: (1, i, 0)))
        else:
            in_specs.append(
                pl.BlockSpec(w, lambda i, _n=len(w): (0,) * _n))
    out_shapes = [jax.ShapeDtypeStruct((N, width_out), jnp.float32)]
    out_specs = [pl.BlockSpec((BR, width_out), lambda i: (i, 0))]
    for w in extra_outs:
        out_shapes.append(jax.ShapeDtypeStruct((N, w), jnp.float32))
        out_specs.append(pl.BlockSpec((BR, w), lambda i: (i, 0)))
    return pl.pallas_call(
        body, grid=grid, in_specs=in_specs,
        out_shape=out_shapes if len(out_shapes) > 1 else out_shapes[0],
        out_specs=out_specs if len(out_specs) > 1 else out_specs[0])


def kernel(x, adj, W1, b1, W2, b2, W3, b3, scale):
    src = adj[0].reshape(NW, NCHUNK, CH)
    dst = adj[1].reshape(NW, NCHUNK, CH)

    enc = _row_call(_enc_in_body,
                    [("rows", IN), ("full", (IN, HID)), ("full", (1, HID))],
                    D1)
    m1 = enc(x, W1[1:], b1.reshape(1, HID))
    agg1 = _seg_sum_partials(m1, src, dst, D1)

    mid1 = _row_call(_mid1_body,
                     [("rows", D1), ("part0", D1), ("part1", D1),
                      ("full", (HID - 1, HID)), ("full", (1, HID))],
                     D2, extra_outs=(8,))
    m2, deg = mid1(m1, agg1, agg1, W2[1:], b2.reshape(1, HID))
    agg2 = _seg_sum_partials(m2, src, dst, D2)

    W3p = jnp.pad(W3[1:], ((0, 0), (0, D3 - OUT - 1)))
    b3p = jnp.pad(b3, (0, D3 - OUT - 1)).reshape(1, D3)
    mid2 = _row_call(_mid2_body,
                     [("rows", D2), ("part0", D2), ("part1", D2),
                      ("rows", 8), ("full", (HID - 1, D3)), ("full", (1, D3))],
                     D3)
    m3 = mid2(m2, agg2, agg2, deg, W3p, b3p)
    agg3 = _seg_sum_partials(m3, src, dst, D3)

    fin = _row_call(_fin_body,
                    [("rows", D3), ("part0", D3), ("part1", D3),
                     ("rows", 8), ("full", (1, 1))],
                    D3)
    zp = fin(m3, agg3, agg3, deg, scale.reshape(1, 1))

    dec = pl.pallas_call(
        _dec_body, grid=(N // BR, N // BR),
        in_specs=[pl.BlockSpec((BR, D3), lambda i, j: (i, 0)),
                  pl.BlockSpec((BR, D3), lambda i, j: (j, 0))],
        out_shape=jax.ShapeDtypeStruct((N, N), jnp.float32),
        out_specs=pl.BlockSpec((BR, BR), lambda i, j: (i, j)))
    adj_pred = dec(zp, zp)
    return (adj_pred, zp[:, :OUT + 1])


# speedup vs baseline: 4.7143x; 1.0348x over previous
import functools

import jax
import jax.numpy as jnp
from jax import lax
from jax.experimental import pallas as pl
from jax.experimental.pallas import tpu as pltpu
from jax.experimental.pallas import tpu_sc as plsc

N = 4096
E = 65536
IN = 128
HID = 256
OUT = 16
R = 2.0
T = 1.0
EPS = 1e-5

NC = 2
NS = 16
CH = 128
NW = NC * NS
NCHUNK = E // (NW * CH)
RPT = N // NS

D1 = 272
D2 = 256
D3 = 32

BR = 512


def _seg_sum_partials(m, src, dst, d):
    mesh = plsc.VectorSubcoreMesh(
        core_axis_name="c", subcore_axis_name="s",
        num_cores=NC, num_subcores=NS)

    def body(m_hbm, src_hbm, dst_hbm, zeros_hbm, out_hbm,
             src_v, dst_v, rows0, agg_sh, sem0):
        cid = lax.axis_index("c")
        sid = lax.axis_index("s")
        w = cid * NS + sid
        pltpu.sync_copy(zeros_hbm, agg_sh.at[pl.ds(sid * RPT, RPT)])
        pltpu.sync_copy(src_hbm.at[w], src_v)
        pltpu.sync_copy(dst_hbm.at[w], dst_v)
        plsc.subcore_barrier()

        def chunk(j, carry):
            pltpu.async_copy(m_hbm.at[src_v.at[j]], rows0, sem0).wait()
            pltpu.sync_copy(rows0, agg_sh.at[dst_v.at[j]], add=True)
            return carry

        lax.fori_loop(0, NCHUNK, chunk, 0)
        plsc.subcore_barrier()
        pltpu.sync_copy(agg_sh.at[pl.ds(sid * RPT, RPT)],
                        out_hbm.at[cid, pl.ds(sid * RPT, RPT)])

    f = pl.kernel(
        body,
        out_type=jax.ShapeDtypeStruct((NC, N, d), jnp.float32),
        mesh=mesh,
        compiler_params=pltpu.CompilerParams(use_tc_tiling_on_sc=False),
        scratch_types=[
            pltpu.VMEM((NCHUNK, CH), jnp.int32),
            pltpu.VMEM((NCHUNK, CH), jnp.int32),
            pltpu.VMEM((CH, d), jnp.float32),
            pltpu.VMEM_SHARED((N, d), jnp.float32),
            pltpu.SemaphoreType.DMA,
        ],
    )
    zeros = jnp.zeros((RPT, d), jnp.float32)
    return f(m, src, dst, zeros)


def _log0(v):
    nrm = jnp.maximum(jnp.sqrt(jnp.sum(v * v, axis=1, keepdims=True)), EPS)
    en = jnp.exp(nrm)
    eni = 1.0 / en
    c = 0.5 * (en + eni)
    s = 0.5 * (en - eni)
    xs = (s / nrm) * v
    nrm2 = jnp.maximum(jnp.sqrt(jnp.sum(xs * xs, axis=1, keepdims=True)), EPS)
    y = jnp.maximum(c, 1.0 + EPS)
    dfac = jnp.log(y + jnp.sqrt(y + 1.0) * jnp.sqrt(y - 1.0))
    return (dfac / nrm2) * xs


def _enc_in_body(x_ref, w_ref, b_ref, o_ref):
    h = _log0(x_ref[...])
    m = jnp.dot(h, w_ref[...], preferred_element_type=jnp.float32) + b_ref[...]
    br = m.shape[0]
    o_ref[...] = jnp.concatenate(
        [m, jnp.ones((br, 1), jnp.float32), jnp.zeros((br, 15), jnp.float32)],
        axis=1)


def _mid1_body(m_ref, a0_ref, a1_ref, w_ref, b_ref, o_ref, deg_ref):
    agg = a0_ref[0] + a1_ref[0]
    deg = jnp.maximum(agg[:, HID:HID + 1], 1.0)
    out = m_ref[:, :HID] + agg[:, :HID] / deg
    h = _log0(out[:, 1:])
    m2 = jnp.dot(h, w_ref[...], preferred_element_type=jnp.float32) + b_ref[...]
    o_ref[...] = m2
    deg_ref[...] = jnp.broadcast_to(deg, (deg.shape[0], 8))


def _mid2_body(m_ref, a0_ref, a1_ref, deg_ref, w_ref, b_ref, o_ref):
    agg = a0_ref[0] + a1_ref[0]
    deg = deg_ref[:, 0:1]
    out = jnp.maximum(m_ref[...] + agg / deg, 0.0)
    h = _log0(out[:, 1:])
    o_ref[...] = (jnp.dot(h, w_ref[...], preferred_element_type=jnp.float32)
                  + b_ref[...])


def _fin_body(m_ref, a0_ref, a1_ref, deg_ref, sc_ref, z_ref):
    agg = a0_ref[0] + a1_ref[0]
    deg = deg_ref[:, 0:1]
    out = jnp.maximum(m_ref[...] + agg / deg, 0.0)
    v = out[:, 1:OUT + 1]
    nrm = jnp.maximum(jnp.sqrt(jnp.sum(v * v, axis=1, keepdims=True)), EPS)
    en = jnp.exp(nrm)
    eni = 1.0 / en
    x0 = 0.5 * (en + eni)
    xs = (0.5 * (en - eni) / nrm) * v
    p = xs / (x0 + 1.0)
    pn = jnp.maximum(jnp.sqrt(jnp.sum(p * p, axis=1, keepdims=True)), 1e-12)
    s = jnp.clip(sc_ref[0, 0], 0.01, 0.999)
    ph = (p / pn) * s
    sq = jnp.sum(ph * ph, axis=1, keepdims=True)
    denom = jnp.maximum(1.0 - sq, EPS)
    z0 = (1.0 + sq) / denom
    zs = 2.0 * ph / denom
    br = z0.shape[0]
    z_ref[...] = jnp.concatenate(
        [z0, zs, jnp.zeros((br, D3 - OUT - 1), jnp.float32)], axis=1)


def _dec_body(zi_ref, zj_ref, o_ref):
    zi = zi_ref[...]
    zj = zj_ref[...]
    col = lax.broadcasted_iota(jnp.int32, (1, D3), 1)
    sgn = jnp.where(col == 0, -1.0, jnp.where(col <= OUT, 1.0, 0.0))
    zim = zi * sgn
    ip = lax.dot_general(zim, zj, (((1,), (1,)), ((), ())),
                         preferred_element_type=jnp.float32)
    y = jnp.maximum(-ip, 1.0 + EPS)
    dh = jnp.log(y + jnp.sqrt(y * y - 1.0))
    t = (R - dh * dh) / T
    p1 = 1.0 / (1.0 + jnp.exp(-t))
    o_ref[...] = 1.0 / (1.0 + jnp.exp(-p1))


def _row_call(body, widths_in, width_out, extra_outs=()):
    grid = (N // BR,)
    in_specs = []
    for kind, w in widths_in:
        if kind == "rows":
            in_specs.append(pl.BlockSpec((BR, w), lambda i: (i, 0)))
        elif kind == "part0":
            in_specs.append(pl.BlockSpec((1, BR, w), lambda i: (0, i, 0)))
        elif kind == "part1":
            in_specs.append(pl.BlockSpec((1, BR, w), lambda i: (1, i, 0)))
        else:
            in_specs.append(
                pl.BlockSpec(w, lambda i, _n=len(w): (0,) * _n))
    out_shapes = [jax.ShapeDtypeStruct((N, width_out), jnp.float32)]
    out_specs = [pl.BlockSpec((BR, width_out), lambda i: (i, 0))]
    for w in extra_outs:
        out_shapes.append(jax.ShapeDtypeStruct((N, w), jnp.float32))
        out_specs.append(pl.BlockSpec((BR, w), lambda i: (i, 0)))
    return pl.pallas_call(
        body, grid=grid, in_specs=in_specs,
        out_shape=out_shapes if len(out_shapes) > 1 else out_shapes[0],
        out_specs=out_specs if len(out_specs) > 1 else out_specs[0])


def kernel(x, adj, W1, b1, W2, b2, W3, b3, scale):
    src = adj[0].reshape(NW, NCHUNK, CH)
    dst = adj[1].reshape(NW, NCHUNK, CH)

    enc = _row_call(_enc_in_body,
                    [("rows", IN), ("full", (IN, HID)), ("full", (1, HID))],
                    D1)
    m1 = enc(x, W1[1:], b1.reshape(1, HID))
    agg1 = _seg_sum_partials(m1, src, dst, D1)

    mid1 = _row_call(_mid1_body,
                     [("rows", D1), ("part0", D1), ("part1", D1),
                      ("full", (HID - 1, HID)), ("full", (1, HID))],
                     D2, extra_outs=(8,))
    m2, deg = mid1(m1, agg1, agg1, W2[1:], b2.reshape(1, HID))
    agg2 = _seg_sum_partials(m2, src, dst, D2)

    W3p = jnp.pad(W3[1:], ((0, 0), (0, D3 - OUT - 1)))
    b3p = jnp.pad(b3, (0, D3 - OUT - 1)).reshape(1, D3)
    mid2 = _row_call(_mid2_body,
                     [("rows", D2), ("part0", D2), ("part1", D2),
                      ("rows", 8), ("full", (HID - 1, D3)), ("full", (1, D3))],
                     D3)
    m3 = mid2(m2, agg2, agg2, deg, W3p, b3p)
    agg3 = _seg_sum_partials(m3, src, dst, D3)

    fin = _row_call(_fin_body,
                    [("rows", D3), ("part0", D3), ("part1", D3),
                     ("rows", 8), ("full", (1, 1))],
                    D3)
    zp = fin(m3, agg3, agg3, deg, scale.reshape(1, 1))

    dec = pl.pallas_call(
        _dec_body, grid=(N // BR, N // BR),
        in_specs=[pl.BlockSpec((BR, D3), lambda i, j: (i, 0)),
                  pl.BlockSpec((BR, D3), lambda i, j: (j, 0))],
        out_shape=jax.ShapeDtypeStruct((N, N), jnp.float32),
        out_specs=pl.BlockSpec((BR, BR), lambda i, j: (i, j)))
    adj_pred = dec(zp, zp)
    return (adj_pred, zp[:, :OUT + 1])
